# Initial kernel scaffold; baseline (speedup 1.0000x reference)
import jax
import jax.numpy as jnp
from jax.experimental import pallas as pl


def kernel(x, embeddings):
    raise NotImplementedError("write your pallas kernel here")


# speedup vs baseline: 1.0231x; 1.0231x over previous
import functools

import jax
import jax.numpy as jnp
from jax import lax
from jax.experimental import pallas as pl
from jax.experimental.pallas import tpu as pltpu
from jax.experimental.pallas import tpu_sc as plsc

VOCAB_SIZE = 1000000
EMBEDDING_DIM = 32
BATCH = 16384
HIST = 50

TOTAL = BATCH * HIST
CHUNK = 128
NUM_WORKERS = 32
ROWS = TOTAL // CHUNK
ROWS_PER_W = ROWS // NUM_WORKERS


def _make_kernel():
  mesh = plsc.VectorSubcoreMesh(core_axis_name="c", subcore_axis_name="s")

  @functools.partial(
      pl.kernel,
      out_type=jax.ShapeDtypeStruct((TOTAL, EMBEDDING_DIM), jnp.float32),
      mesh=mesh,
      scratch_types=[
          pltpu.VMEM((ROWS_PER_W, CHUNK), jnp.int32),
          pltpu.VMEM((CHUNK, EMBEDDING_DIM), jnp.float32),
          pltpu.SemaphoreType.DMA,
      ],
      compiler_params=pltpu.CompilerParams(use_tc_tiling_on_sc=False),
  )
  def gather_kernel(table_hbm, idx_hbm, out_hbm, idx_v, rows_v, sem):
    wid = lax.axis_index("s") * 2 + lax.axis_index("c")
    base = wid * ROWS_PER_W
    pltpu.sync_copy(idx_hbm.at[pl.ds(base, ROWS_PER_W)], idx_v)

    def chunk_body(j, carry):
      pltpu.async_copy(table_hbm.at[idx_v.at[j]], rows_v, sem).wait()
      pltpu.sync_copy(rows_v, out_hbm.at[pl.ds((base + j) * CHUNK, CHUNK)])
      return carry

    lax.fori_loop(0, ROWS_PER_W, chunk_body, 0)

  return gather_kernel


_gather = _make_kernel()


@jax.jit
def kernel(x, embeddings):
  idx2d = x.reshape(ROWS, CHUNK).astype(jnp.int32)
  out = _gather(embeddings, idx2d)
  return out.reshape(BATCH, HIST, EMBEDDING_DIM)


# speedup vs baseline: 1.1101x; 1.0850x over previous
import functools

import jax
import jax.numpy as jnp
from jax import lax
from jax.experimental import pallas as pl
from jax.experimental.pallas import tpu as pltpu
from jax.experimental.pallas import tpu_sc as plsc

VOCAB_SIZE = 1000000
EMBEDDING_DIM = 32
BATCH = 16384
HIST = 50

TOTAL = BATCH * HIST
CHUNK = 128
NUM_WORKERS = 32
ROWS = TOTAL // CHUNK
ROWS_PER_W = ROWS // NUM_WORKERS
GROUP = 8
NGROUPS = ROWS_PER_W // GROUP
GROUP_ROWS = GROUP * CHUNK


def _make_kernel():
  mesh = plsc.VectorSubcoreMesh(core_axis_name="c", subcore_axis_name="s")

  @functools.partial(
      pl.kernel,
      out_type=jax.ShapeDtypeStruct((TOTAL, EMBEDDING_DIM), jnp.float32),
      mesh=mesh,
      scratch_types=[
          pltpu.VMEM((ROWS_PER_W, CHUNK), jnp.int32),
          pltpu.VMEM((2, GROUP_ROWS, EMBEDDING_DIM), jnp.float32),
          pltpu.SemaphoreType.DMA,
          pltpu.SemaphoreType.DMA,
      ],
      compiler_params=pltpu.CompilerParams(use_tc_tiling_on_sc=False),
  )
  def gather_kernel(table_hbm, idx_hbm, out_hbm, idx_v, rows_v, gsem, osem):
    wid = lax.axis_index("s") * 2 + lax.axis_index("c")
    base = wid * ROWS_PER_W
    pltpu.sync_copy(idx_hbm.at[pl.ds(base, ROWS_PER_W)], idx_v)

    def group_body(g, carry):
      buf = lax.rem(g, 2)

      @pl.when(g >= 2)
      def _():
        pltpu.make_async_copy(
            rows_v.at[buf],
            out_hbm.at[pl.ds(0, GROUP_ROWS)],
            osem,
        ).wait()

      for b in range(GROUP):
        pltpu.async_copy(
            table_hbm.at[idx_v.at[g * GROUP + b]],
            rows_v.at[buf, pl.ds(b * CHUNK, CHUNK)],
            gsem,
        )
      for b in range(GROUP):
        pltpu.make_async_copy(
            table_hbm.at[idx_v.at[g * GROUP + b]],
            rows_v.at[buf, pl.ds(b * CHUNK, CHUNK)],
            gsem,
        ).wait()

      pltpu.async_copy(
          rows_v.at[buf],
          out_hbm.at[pl.ds((base + g * GROUP) * CHUNK, GROUP_ROWS)],
          osem,
      )
      return carry

    lax.fori_loop(0, NGROUPS, group_body, 0)

    for _ in range(2):
      pltpu.make_async_copy(
          rows_v.at[0],
          out_hbm.at[pl.ds(0, GROUP_ROWS)],
          osem,
      ).wait()

  return gather_kernel


_gather = _make_kernel()


@jax.jit
def kernel(x, embeddings):
  idx2d = x.reshape(ROWS, CHUNK).astype(jnp.int32)
  out = _gather(embeddings, idx2d)
  return out.reshape(BATCH, HIST, EMBEDDING_DIM)
